# baseline (device time: 75393 ns/iter reference)
import jax
import jax.numpy as jnp
from jax import lax
from jax.experimental import pallas as pl
from jax.experimental.pallas import tpu as pltpu

N_DEV = 4
B, Sq, D = 2, 256, 768
HQ_LOCAL, DH = 8, 64
DL = HQ_LOCAL * DH
SCALE = 0.125


def kernel(x, Wq, Wo, Wk, Wv):
    def body(x_ref, wq_ref, wo_ref, wk_ref, wv_ref, out_ref,
             attn_ref, comm_ref, send_sems, recv_sems):
        my = lax.axis_index("i")
        left = (my - 1) % N_DEV
        right = (my + 1) % N_DEV

        barrier_sem = pltpu.get_barrier_semaphore()
        for nbr in [left, right]:
            pl.semaphore_signal(
                barrier_sem, inc=1,
                device_id=(nbr,), device_id_type=pl.DeviceIdType.MESH,
            )
        pl.semaphore_wait(barrier_sem, 2)

        x2d = x_ref[:].reshape(B * Sq, D)
        q = jnp.dot(x2d, wq_ref[:], preferred_element_type=jnp.float32)
        k = jnp.dot(x2d, wk_ref[:], preferred_element_type=jnp.float32)
        v = jnp.dot(x2d, wv_ref[:], preferred_element_type=jnp.float32)

        for b in range(B):
            r0 = b * Sq
            for h in range(HQ_LOCAL):
                c0 = h * DH
                qb = q[r0:r0 + Sq, c0:c0 + DH]
                kb = k[r0:r0 + Sq, c0:c0 + DH]
                vb = v[r0:r0 + Sq, c0:c0 + DH]
                s = lax.dot_general(
                    qb, kb, (((1,), (1,)), ((), ())),
                    preferred_element_type=jnp.float32,
                ) * SCALE
                m = jnp.max(s, axis=-1, keepdims=True)
                p = jnp.exp(s - m)
                l = jnp.sum(p, axis=-1, keepdims=True)
                o = jnp.dot(p, vb, preferred_element_type=jnp.float32) / l
                attn_ref[r0:r0 + Sq, c0:c0 + DH] = o

        partial = jnp.dot(attn_ref[:], wo_ref[:],
                          preferred_element_type=jnp.float32)
        comm_ref[0] = partial
        acc = partial

        for h in range(N_DEV - 1):
            rdma = pltpu.make_async_remote_copy(
                src_ref=comm_ref.at[h],
                dst_ref=comm_ref.at[h + 1],
                send_sem=send_sems.at[h],
                recv_sem=recv_sems.at[h],
                device_id=(right,),
                device_id_type=pl.DeviceIdType.MESH,
            )
            rdma.start()
            rdma.wait()
            acc = acc + comm_ref[h + 1]

        out_ref[:] = acc.reshape(B, Sq, D)

    return pl.pallas_call(
        body,
        out_shape=jax.ShapeDtypeStruct((B, Sq, D), jnp.float32),
        in_specs=[
            pl.BlockSpec(memory_space=pltpu.VMEM),
            pl.BlockSpec(memory_space=pltpu.VMEM),
            pl.BlockSpec(memory_space=pltpu.VMEM),
            pl.BlockSpec(memory_space=pltpu.VMEM),
            pl.BlockSpec(memory_space=pltpu.VMEM),
        ],
        out_specs=pl.BlockSpec(memory_space=pltpu.VMEM),
        scratch_shapes=[
            pltpu.VMEM((B * Sq, DL), jnp.float32),
            pltpu.VMEM((N_DEV, B * Sq, D), jnp.float32),
            pltpu.SemaphoreType.DMA((N_DEV - 1,)),
            pltpu.SemaphoreType.DMA((N_DEV - 1,)),
        ],
        compiler_params=pltpu.CompilerParams(collective_id=0),
    )(x, Wq, Wo, Wk, Wv)


# device time: 37147 ns/iter; 2.0296x vs baseline; 2.0296x over previous
import jax
import jax.numpy as jnp
from jax import lax
from jax.experimental import pallas as pl
from jax.experimental.pallas import tpu as pltpu

N_DEV = 4
B, Sq, D = 2, 256, 768
HQ, DH = 8, 64
DL = HQ * DH
CH = (B * Sq) // N_DEV
SCALE = 0.125


def kernel(x, Wq, Wo, Wk, Wv):
    def body(x_ref, wq_ref, wo_ref, wk_ref, wv_ref, out_ref,
             q_ref, k_ref, v_ref, attnc_ref, stage_ref, recv_ref, ag_ref,
             sc_send_sems, sc_recv_sems, ag_send_sems, ag_recv_sems):
        my = lax.axis_index("i")
        others = [(my + 1) % N_DEV, (my + 2) % N_DEV, (my + 3) % N_DEV]

        barrier_sem = pltpu.get_barrier_semaphore()
        for nbr in others:
            pl.semaphore_signal(
                barrier_sem, inc=1,
                device_id=(nbr,), device_id_type=pl.DeviceIdType.MESH,
            )

        x2d = x_ref[:].reshape(B * Sq, D)
        q_ref[:] = jnp.dot(x2d, wq_ref[:], preferred_element_type=jnp.float32)
        k_ref[:] = jnp.dot(x2d, wk_ref[:], preferred_element_type=jnp.float32)
        v_ref[:] = jnp.dot(x2d, wv_ref[:], preferred_element_type=jnp.float32)

        pl.semaphore_wait(barrier_sem, N_DEV - 1)

        sends = []
        for kk in range(N_DEV):
            c = (my + 1 + kk) % N_DEV
            qc = q_ref[pl.ds(c * CH, CH), :]
            b0 = (c // 2) * Sq
            kb = k_ref[pl.ds(b0, Sq), :]
            vb = v_ref[pl.ds(b0, Sq), :]
            for h in range(HQ):
                c0 = h * DH
                s = lax.dot_general(
                    qc[:, c0:c0 + DH], kb[:, c0:c0 + DH],
                    (((1,), (1,)), ((), ())),
                    preferred_element_type=jnp.float32,
                ) * SCALE
                m = jnp.max(s, axis=-1, keepdims=True)
                p = jnp.exp(s - m)
                l = jnp.sum(p, axis=-1, keepdims=True)
                o = jnp.dot(p, vb[:, c0:c0 + DH],
                            preferred_element_type=jnp.float32) / l
                attnc_ref[:, c0:c0 + DH] = o
            stage_ref[kk] = jnp.dot(attnc_ref[:], wo_ref[:],
                                    preferred_element_type=jnp.float32)
            if kk < N_DEV - 1:
                r = (N_DEV - 1) - kk
                rdma = pltpu.make_async_remote_copy(
                    src_ref=stage_ref.at[kk],
                    dst_ref=recv_ref.at[r],
                    send_sem=sc_send_sems.at[kk],
                    recv_sem=sc_recv_sems.at[r],
                    device_id=(c,),
                    device_id_type=pl.DeviceIdType.MESH,
                )
                rdma.start()
                sends.append(rdma)

        for r in range(1, N_DEV):
            rd = pltpu.make_async_remote_copy(
                src_ref=stage_ref.at[0],
                dst_ref=recv_ref.at[r],
                send_sem=sc_send_sems.at[0],
                recv_sem=sc_recv_sems.at[r],
                device_id=(my,),
                device_id_type=pl.DeviceIdType.MESH,
            )
            rd.wait_recv()
        red = stage_ref[N_DEV - 1] + recv_ref[1] + recv_ref[2] + recv_ref[3]
        ag_ref[my] = red

        ag_sends = []
        for t in range(1, N_DEV):
            peer = (my + t) % N_DEV
            rdma = pltpu.make_async_remote_copy(
                src_ref=ag_ref.at[my],
                dst_ref=ag_ref.at[my],
                send_sem=ag_send_sems.at[t - 1],
                recv_sem=ag_recv_sems.at[N_DEV - t],
                device_id=(peer,),
                device_id_type=pl.DeviceIdType.MESH,
            )
            rdma.start()
            ag_sends.append(rdma)

        for s_ in sends:
            s_.wait_send()

        for s in range(1, N_DEV):
            owner = (my + s) % N_DEV
            rd = pltpu.make_async_remote_copy(
                src_ref=ag_ref.at[0],
                dst_ref=ag_ref.at[owner],
                send_sem=ag_send_sems.at[0],
                recv_sem=ag_recv_sems.at[s],
                device_id=(my,),
                device_id_type=pl.DeviceIdType.MESH,
            )
            rd.wait_recv()

        for o in range(N_DEV):
            s0 = (o % 2) * CH
            out_ref[o // 2, s0:s0 + CH, :] = ag_ref[o]

        for s_ in ag_sends:
            s_.wait_send()

    return pl.pallas_call(
        body,
        out_shape=jax.ShapeDtypeStruct((B, Sq, D), jnp.float32),
        in_specs=[
            pl.BlockSpec(memory_space=pltpu.VMEM),
            pl.BlockSpec(memory_space=pltpu.VMEM),
            pl.BlockSpec(memory_space=pltpu.VMEM),
            pl.BlockSpec(memory_space=pltpu.VMEM),
            pl.BlockSpec(memory_space=pltpu.VMEM),
        ],
        out_specs=pl.BlockSpec(memory_space=pltpu.VMEM),
        scratch_shapes=[
            pltpu.VMEM((B * Sq, DL), jnp.float32),
            pltpu.VMEM((B * Sq, DL), jnp.float32),
            pltpu.VMEM((B * Sq, DL), jnp.float32),
            pltpu.VMEM((CH, DL), jnp.float32),
            pltpu.VMEM((N_DEV, CH, D), jnp.float32),
            pltpu.VMEM((N_DEV, CH, D), jnp.float32),
            pltpu.VMEM((N_DEV, CH, D), jnp.float32),
            pltpu.SemaphoreType.DMA((N_DEV - 1,)),
            pltpu.SemaphoreType.DMA((N_DEV,)),
            pltpu.SemaphoreType.DMA((N_DEV - 1,)),
            pltpu.SemaphoreType.DMA((N_DEV,)),
        ],
        compiler_params=pltpu.CompilerParams(collective_id=0),
    )(x, Wq, Wo, Wk, Wv)


# device time: 28169 ns/iter; 2.6765x vs baseline; 1.3187x over previous
import jax
import jax.numpy as jnp
from jax import lax
from jax.experimental import pallas as pl
from jax.experimental.pallas import tpu as pltpu

N_DEV = 4
B, Sq, D = 2, 256, 768
HQ, DH = 8, 64
DL = HQ * DH
CH = (B * Sq) // N_DEV
SCALE = 0.125
BF = jnp.bfloat16


def kernel(x, Wq, Wo, Wk, Wv):
    def body(x_ref, wq_ref, wo_ref, wk_ref, wv_ref, out_ref,
             q_ref, k_ref, v_ref, attnc_ref, stage_ref, own_ref,
             recv_ref, ag_ref,
             sc_send_sems, sc_recv_sems, ag_send_sems, ag_recv_sems):
        my = lax.axis_index("i")
        others = [(my + 1) % N_DEV, (my + 2) % N_DEV, (my + 3) % N_DEV]

        barrier_sem = pltpu.get_barrier_semaphore()
        for nbr in others:
            pl.semaphore_signal(
                barrier_sem, inc=1,
                device_id=(nbr,), device_id_type=pl.DeviceIdType.MESH,
            )

        xb = x_ref[:].reshape(B * Sq, D).astype(BF)
        q_ref[:] = jnp.dot(xb, wq_ref[:].astype(BF),
                           preferred_element_type=jnp.float32).astype(BF)
        k_ref[:] = jnp.dot(xb, wk_ref[:].astype(BF),
                           preferred_element_type=jnp.float32).astype(BF)
        v_ref[:] = jnp.dot(xb, wv_ref[:].astype(BF),
                           preferred_element_type=jnp.float32).astype(BF)
        wo_b = wo_ref[:].astype(BF)

        pl.semaphore_wait(barrier_sem, N_DEV - 1)

        sends = []
        for kk in range(N_DEV):
            c = (my + 1 + kk) % N_DEV
            qc = q_ref[pl.ds(c * CH, CH), :]
            b0 = (c // 2) * Sq
            kb = k_ref[pl.ds(b0, Sq), :]
            vb = v_ref[pl.ds(b0, Sq), :]
            for h in range(HQ):
                c0 = h * DH
                s = lax.dot_general(
                    qc[:, c0:c0 + DH], kb[:, c0:c0 + DH],
                    (((1,), (1,)), ((), ())),
                    preferred_element_type=jnp.float32,
                ) * SCALE
                m = jnp.max(s, axis=-1, keepdims=True)
                p = jnp.exp(s - m)
                l = jnp.sum(p, axis=-1, keepdims=True)
                o = jnp.dot(p.astype(BF), vb[:, c0:c0 + DH],
                            preferred_element_type=jnp.float32) / l
                attnc_ref[:, c0:c0 + DH] = o.astype(BF)
            partial = jnp.dot(attnc_ref[:], wo_b,
                              preferred_element_type=jnp.float32)
            if kk < N_DEV - 1:
                stage_ref[kk] = partial.astype(BF)
                r = (N_DEV - 1) - kk
                rdma = pltpu.make_async_remote_copy(
                    src_ref=stage_ref.at[kk],
                    dst_ref=recv_ref.at[r],
                    send_sem=sc_send_sems.at[kk],
                    recv_sem=sc_recv_sems.at[r],
                    device_id=(c,),
                    device_id_type=pl.DeviceIdType.MESH,
                )
                rdma.start()
                sends.append(rdma)
            else:
                own_ref[:] = partial

        for r in range(1, N_DEV):
            rd = pltpu.make_async_remote_copy(
                src_ref=stage_ref.at[0],
                dst_ref=recv_ref.at[r],
                send_sem=sc_send_sems.at[0],
                recv_sem=sc_recv_sems.at[r],
                device_id=(my,),
                device_id_type=pl.DeviceIdType.MESH,
            )
            rd.wait_recv()
        red = (own_ref[:]
               + recv_ref[1].astype(jnp.float32)
               + recv_ref[2].astype(jnp.float32)
               + recv_ref[3].astype(jnp.float32))
        ag_ref[my] = red.astype(BF)

        ag_sends = []
        for t in range(1, N_DEV):
            peer = (my + t) % N_DEV
            rdma = pltpu.make_async_remote_copy(
                src_ref=ag_ref.at[my],
                dst_ref=ag_ref.at[my],
                send_sem=ag_send_sems.at[t - 1],
                recv_sem=ag_recv_sems.at[N_DEV - t],
                device_id=(peer,),
                device_id_type=pl.DeviceIdType.MESH,
            )
            rdma.start()
            ag_sends.append(rdma)

        for s_ in sends:
            s_.wait_send()

        for s in range(1, N_DEV):
            owner = (my + s) % N_DEV
            rd = pltpu.make_async_remote_copy(
                src_ref=ag_ref.at[0],
                dst_ref=ag_ref.at[owner],
                send_sem=ag_send_sems.at[0],
                recv_sem=ag_recv_sems.at[s],
                device_id=(my,),
                device_id_type=pl.DeviceIdType.MESH,
            )
            rd.wait_recv()

        for o in range(N_DEV):
            s0 = (o % 2) * CH
            out_ref[o // 2, s0:s0 + CH, :] = ag_ref[o].astype(jnp.float32)

        for s_ in ag_sends:
            s_.wait_send()

    return pl.pallas_call(
        body,
        out_shape=jax.ShapeDtypeStruct((B, Sq, D), jnp.float32),
        in_specs=[
            pl.BlockSpec(memory_space=pltpu.VMEM),
            pl.BlockSpec(memory_space=pltpu.VMEM),
            pl.BlockSpec(memory_space=pltpu.VMEM),
            pl.BlockSpec(memory_space=pltpu.VMEM),
            pl.BlockSpec(memory_space=pltpu.VMEM),
        ],
        out_specs=pl.BlockSpec(memory_space=pltpu.VMEM),
        scratch_shapes=[
            pltpu.VMEM((B * Sq, DL), BF),
            pltpu.VMEM((B * Sq, DL), BF),
            pltpu.VMEM((B * Sq, DL), BF),
            pltpu.VMEM((CH, DL), BF),
            pltpu.VMEM((N_DEV - 1, CH, D), BF),
            pltpu.VMEM((CH, D), jnp.float32),
            pltpu.VMEM((N_DEV, CH, D), BF),
            pltpu.VMEM((N_DEV, CH, D), BF),
            pltpu.SemaphoreType.DMA((N_DEV - 1,)),
            pltpu.SemaphoreType.DMA((N_DEV,)),
            pltpu.SemaphoreType.DMA((N_DEV - 1,)),
            pltpu.SemaphoreType.DMA((N_DEV,)),
        ],
        compiler_params=pltpu.CompilerParams(collective_id=0),
    )(x, Wq, Wo, Wk, Wv)


# device time: 27691 ns/iter; 2.7227x vs baseline; 1.0173x over previous
import jax
import jax.numpy as jnp
from jax import lax
from jax.experimental import pallas as pl
from jax.experimental.pallas import tpu as pltpu

N_DEV = 4
B, Sq, D = 2, 256, 768
HQ, DH = 8, 64
DL = HQ * DH
CH = (B * Sq) // N_DEV
SCALE = 0.125
BF = jnp.bfloat16


def kernel(x, Wq, Wo, Wk, Wv):
    def body(x_ref, wq_ref, wo_ref, wk_ref, wv_ref, out_ref,
             x_v, wq_v, wo_v, wk_v, wv_v,
             q_ref, k_ref, v_ref, attnc_ref, pb_ref, stage_ref, own_ref,
             recv_ref, ag_ref,
             in_sems, sc_send_sems, sc_recv_sems, ag_send_sems,
             ag_recv_sems):
        my = lax.axis_index("i")
        others = [(my + 1) % N_DEV, (my + 2) % N_DEV, (my + 3) % N_DEV]

        cps = []
        for i, (hbm, vmem) in enumerate([(x_ref, x_v), (wq_ref, wq_v),
                                         (wk_ref, wk_v), (wv_ref, wv_v),
                                         (wo_ref, wo_v)]):
            cp = pltpu.make_async_copy(hbm, vmem, in_sems.at[i])
            cp.start()
            cps.append(cp)

        barrier_sem = pltpu.get_barrier_semaphore()
        for nbr in others:
            pl.semaphore_signal(
                barrier_sem, inc=1,
                device_id=(nbr,), device_id_type=pl.DeviceIdType.MESH,
            )

        cps[0].wait()
        xb = x_v[:].reshape(B * Sq, D).astype(BF)
        cps[1].wait()
        q_ref[:] = (jnp.dot(xb, wq_v[:].astype(BF),
                            preferred_element_type=jnp.float32)
                    * SCALE).astype(BF)
        cps[2].wait()
        k_ref[:] = jnp.dot(xb, wk_v[:].astype(BF),
                           preferred_element_type=jnp.float32).astype(BF)
        cps[3].wait()
        v_ref[:] = jnp.dot(xb, wv_v[:].astype(BF),
                           preferred_element_type=jnp.float32).astype(BF)
        cps[4].wait()
        wo_b = wo_v[:].astype(BF)

        rows = lax.broadcasted_iota(jnp.int32, (HQ * Sq, HQ), 0)
        cols = lax.broadcasted_iota(jnp.int32, (HQ * Sq, HQ), 1)
        blockones = jnp.where(rows // Sq == cols, 1.0, 0.0).astype(BF)

        pl.semaphore_wait(barrier_sem, N_DEV - 1)

        sends = []
        for kk in range(N_DEV):
            c = (my + 1 + kk) % N_DEV
            qc = q_ref[pl.ds(c * CH, CH), :]
            b0 = (c // 2) * Sq
            kb = k_ref[pl.ds(b0, Sq), :]
            vb = v_ref[pl.ds(b0, Sq), :]
            for h in range(HQ):
                s = lax.dot_general(
                    qc[:, h * DH:(h + 1) * DH], kb[:, h * DH:(h + 1) * DH],
                    (((1,), (1,)), ((), ())),
                    preferred_element_type=jnp.float32,
                )
                pb_ref[:, h * Sq:(h + 1) * Sq] = jnp.exp(s).astype(BF)
            pb = pb_ref[:]
            linv = 1.0 / jnp.dot(pb, blockones,
                                 preferred_element_type=jnp.float32)
            for h in range(HQ):
                c0 = h * DH
                o = jnp.dot(pb[:, h * Sq:(h + 1) * Sq], vb[:, c0:c0 + DH],
                            preferred_element_type=jnp.float32)
                attnc_ref[:, c0:c0 + DH] = (o * linv[:, h:h + 1]).astype(BF)
            partial = jnp.dot(attnc_ref[:], wo_b,
                              preferred_element_type=jnp.float32)
            if kk < N_DEV - 1:
                stage_ref[kk] = partial.astype(BF)
                r = (N_DEV - 1) - kk
                rdma = pltpu.make_async_remote_copy(
                    src_ref=stage_ref.at[kk],
                    dst_ref=recv_ref.at[r],
                    send_sem=sc_send_sems.at[kk],
                    recv_sem=sc_recv_sems.at[r],
                    device_id=(c,),
                    device_id_type=pl.DeviceIdType.MESH,
                )
                rdma.start()
                sends.append(rdma)
            else:
                own_ref[:] = partial

        acc = own_ref[:]
        for r in (3, 2, 1):
            rd = pltpu.make_async_remote_copy(
                src_ref=stage_ref.at[0],
                dst_ref=recv_ref.at[r],
                send_sem=sc_send_sems.at[0],
                recv_sem=sc_recv_sems.at[r],
                device_id=(my,),
                device_id_type=pl.DeviceIdType.MESH,
            )
            rd.wait_recv()
            acc = acc + recv_ref[r].astype(jnp.float32)
        ag_ref[my] = acc.astype(BF)

        ag_sends = []
        for t in range(1, N_DEV):
            peer = (my + t) % N_DEV
            rdma = pltpu.make_async_remote_copy(
                src_ref=ag_ref.at[my],
                dst_ref=ag_ref.at[my],
                send_sem=ag_send_sems.at[t - 1],
                recv_sem=ag_recv_sems.at[N_DEV - t],
                device_id=(peer,),
                device_id_type=pl.DeviceIdType.MESH,
            )
            rdma.start()
            ag_sends.append(rdma)

        for s_ in sends:
            s_.wait_send()

        for s in range(1, N_DEV):
            owner = (my + s) % N_DEV
            rd = pltpu.make_async_remote_copy(
                src_ref=ag_ref.at[0],
                dst_ref=ag_ref.at[owner],
                send_sem=ag_send_sems.at[0],
                recv_sem=ag_recv_sems.at[s],
                device_id=(my,),
                device_id_type=pl.DeviceIdType.MESH,
            )
            rd.wait_recv()

        for o in range(N_DEV):
            s0 = (o % 2) * CH
            out_ref[o // 2, s0:s0 + CH, :] = ag_ref[o].astype(jnp.float32)

        for s_ in ag_sends:
            s_.wait_send()

    return pl.pallas_call(
        body,
        out_shape=jax.ShapeDtypeStruct((B, Sq, D), jnp.float32),
        in_specs=[
            pl.BlockSpec(memory_space=pltpu.MemorySpace.HBM),
            pl.BlockSpec(memory_space=pltpu.MemorySpace.HBM),
            pl.BlockSpec(memory_space=pltpu.MemorySpace.HBM),
            pl.BlockSpec(memory_space=pltpu.MemorySpace.HBM),
            pl.BlockSpec(memory_space=pltpu.MemorySpace.HBM),
        ],
        out_specs=pl.BlockSpec(memory_space=pltpu.VMEM),
        scratch_shapes=[
            pltpu.VMEM((B, Sq, D), jnp.float32),
            pltpu.VMEM((D, DL), jnp.float32),
            pltpu.VMEM((DL, D), jnp.float32),
            pltpu.VMEM((D, DL), jnp.float32),
            pltpu.VMEM((D, DL), jnp.float32),
            pltpu.VMEM((B * Sq, DL), BF),
            pltpu.VMEM((B * Sq, DL), BF),
            pltpu.VMEM((B * Sq, DL), BF),
            pltpu.VMEM((CH, DL), BF),
            pltpu.VMEM((CH, HQ * Sq), BF),
            pltpu.VMEM((N_DEV - 1, CH, D), BF),
            pltpu.VMEM((CH, D), jnp.float32),
            pltpu.VMEM((N_DEV, CH, D), BF),
            pltpu.VMEM((N_DEV, CH, D), BF),
            pltpu.SemaphoreType.DMA((5,)),
            pltpu.SemaphoreType.DMA((N_DEV - 1,)),
            pltpu.SemaphoreType.DMA((N_DEV,)),
            pltpu.SemaphoreType.DMA((N_DEV - 1,)),
            pltpu.SemaphoreType.DMA((N_DEV,)),
        ],
        compiler_params=pltpu.CompilerParams(collective_id=0),
    )(x, Wq, Wo, Wk, Wv)
